# baseline (speedup 1.0000x reference)
import jax
import jax.numpy as jnp
from jax.experimental import pallas as pl
from jax.experimental.pallas import tpu as pltpu

_K = 512


def _body(lens_ref, x_ref, w1_ref, b1_ref, w2_ref,
          act_ref, lp_ref, en_ref, sc_ref, carry_ref):
    b = pl.program_id(0)
    c = pl.program_id(1)
    nc = pl.num_programs(1)
    K, NC = sc_ref.shape
    Hh = carry_ref.shape[1]
    len_b = lens_ref[b]

    @pl.when(c * K <= len_b - 1)
    def _compute():
        x = x_ref[0]
        r = jnp.dot(x, w1_ref[...], preferred_element_type=jnp.float32)
        p_part = r[:, :Hh]
        q_part = r[:, Hh:]
        p_shift = jnp.concatenate([carry_ref[...], p_part[:-1]], axis=0)
        h = jnp.maximum(p_shift + q_part + b1_ref[...], 0.0)
        v = jnp.dot(h, w2_ref[...], preferred_element_type=jnp.float32)
        lane = jax.lax.broadcasted_iota(jnp.int32, (K, NC), 1)
        sc_ref[...] = jnp.where(lane == c, v, sc_ref[...])
        carry_ref[...] = p_part[-1:, :]

    @pl.when(c == nc - 1)
    def _finalize():
        s_all = sc_ref[...]
        g = (jax.lax.broadcasted_iota(jnp.int32, (K, NC), 0)
             + K * jax.lax.broadcasted_iota(jnp.int32, (K, NC), 1))
        valid = (g >= 1) & (g <= len_b - 1)
        s_m = jnp.where(valid, s_all, -jnp.inf)
        m = jnp.max(s_m)
        e = jnp.where(valid, jnp.exp(s_all - m), 0.0)
        l = jnp.sum(e)
        s_clean = jnp.where(valid, s_all, 0.0)
        t = jnp.sum(e * s_clean)
        cand = jnp.where(s_m == m, g, jnp.int32(2**30))
        gmin = jnp.min(cand)
        empty = len_b <= 1
        nan = jnp.float32(jnp.nan)
        logl = jnp.log(l)
        act_ref[0, 0, 0] = jnp.where(empty, 0, gmin - 1)
        lp_ref[0, 0, 0] = jnp.where(empty, nan, -logl)
        en_ref[0, 0, 0] = jnp.where(empty, nan, m + logl - t / l)


def kernel(sequence_embedding, sentence_lengths, W1, b1, W2, b2):
    B, S, D = sequence_embedding.shape
    H = W1.shape[0]
    K = _K
    NC = S // K
    w1cat = jnp.concatenate([W1[:, :D].T, W1[:, D:].T], axis=1)
    b1r = b1.reshape(1, H)
    w2c = W2.reshape(H, 1)

    grid_spec = pltpu.PrefetchScalarGridSpec(
        num_scalar_prefetch=1,
        grid=(B, NC),
        in_specs=[
            pl.BlockSpec(
                (1, K, D),
                lambda b, c, lens: (b, jnp.minimum(c, jnp.maximum(lens[b] - 1, 0) // K), 0)),
            pl.BlockSpec((D, 2 * H), lambda b, c, lens: (0, 0)),
            pl.BlockSpec((1, H), lambda b, c, lens: (0, 0)),
            pl.BlockSpec((H, 1), lambda b, c, lens: (0, 0)),
        ],
        out_specs=[
            pl.BlockSpec((1, 1, 1), lambda b, c, lens: (b, 0, 0),
                         memory_space=pltpu.SMEM),
            pl.BlockSpec((1, 1, 1), lambda b, c, lens: (b, 0, 0),
                         memory_space=pltpu.SMEM),
            pl.BlockSpec((1, 1, 1), lambda b, c, lens: (b, 0, 0),
                         memory_space=pltpu.SMEM),
        ],
        scratch_shapes=[
            pltpu.VMEM((K, NC), jnp.float32),
            pltpu.VMEM((1, H), jnp.float32),
        ],
    )
    act, lp, en = pl.pallas_call(
        _body,
        grid_spec=grid_spec,
        out_shape=[
            jax.ShapeDtypeStruct((B, 1, 1), jnp.int32),
            jax.ShapeDtypeStruct((B, 1, 1), jnp.float32),
            jax.ShapeDtypeStruct((B, 1, 1), jnp.float32),
        ],
    )(sentence_lengths, sequence_embedding, w1cat, b1r, w2c)
    return act[:, 0, 0], lp[:, 0, 0], en[:, 0, 0]


# speedup vs baseline: 1.0008x; 1.0008x over previous
import jax
import jax.numpy as jnp
from jax.experimental import pallas as pl
from jax.experimental.pallas import tpu as pltpu

_K = 512


def _body(lens_ref, x_ref, w1_ref, b1_ref, w2_ref,
          act_ref, lp_ref, en_ref, sc_ref, carry_ref):
    b = pl.program_id(0)
    c = pl.program_id(1)
    nc = pl.num_programs(1)
    K, NC = sc_ref.shape
    Hh = carry_ref.shape[1]
    len_b = lens_ref[b]

    @pl.when(c * K <= len_b - 1)
    def _compute():
        x = x_ref[0].astype(jnp.bfloat16)
        r = jnp.dot(x, w1_ref[...], preferred_element_type=jnp.float32)
        p_part = r[:, :Hh]
        q_part = r[:, Hh:]
        p_shift = jnp.concatenate([carry_ref[...], p_part[:-1]], axis=0)
        h = jnp.maximum(p_shift + q_part + b1_ref[...], 0.0).astype(jnp.bfloat16)
        v = jnp.dot(h, w2_ref[...], preferred_element_type=jnp.float32)
        lane = jax.lax.broadcasted_iota(jnp.int32, (K, NC), 1)
        sc_ref[...] = jnp.where(lane == c, v, sc_ref[...])
        carry_ref[...] = p_part[-1:, :]

    @pl.when(c == nc - 1)
    def _finalize():
        s_all = sc_ref[...]
        g = (jax.lax.broadcasted_iota(jnp.int32, (K, NC), 0)
             + K * jax.lax.broadcasted_iota(jnp.int32, (K, NC), 1))
        valid = (g >= 1) & (g <= len_b - 1)
        s_m = jnp.where(valid, s_all, -jnp.inf)
        m = jnp.max(s_m)
        e = jnp.where(valid, jnp.exp(s_all - m), 0.0)
        l = jnp.sum(e)
        s_clean = jnp.where(valid, s_all, 0.0)
        t = jnp.sum(e * s_clean)
        cand = jnp.where(s_m == m, g, jnp.int32(2**30))
        gmin = jnp.min(cand)
        empty = len_b <= 1
        nan = jnp.float32(jnp.nan)
        logl = jnp.log(l)
        act_ref[0, 0, 0] = jnp.where(empty, 0, gmin - 1)
        lp_ref[0, 0, 0] = jnp.where(empty, nan, -logl)
        en_ref[0, 0, 0] = jnp.where(empty, nan, m + logl - t / l)


def kernel(sequence_embedding, sentence_lengths, W1, b1, W2, b2):
    B, S, D = sequence_embedding.shape
    H = W1.shape[0]
    K = _K
    NC = S // K
    w1cat = jnp.concatenate([W1[:, :D].T, W1[:, D:].T], axis=1).astype(jnp.bfloat16)
    b1r = b1.reshape(1, H)
    w2c = W2.reshape(H, 1).astype(jnp.bfloat16)

    grid_spec = pltpu.PrefetchScalarGridSpec(
        num_scalar_prefetch=1,
        grid=(B, NC),
        in_specs=[
            pl.BlockSpec(
                (1, K, D),
                lambda b, c, lens: (b, jnp.minimum(c, jnp.maximum(lens[b] - 1, 0) // K), 0)),
            pl.BlockSpec((D, 2 * H), lambda b, c, lens: (0, 0)),
            pl.BlockSpec((1, H), lambda b, c, lens: (0, 0)),
            pl.BlockSpec((H, 1), lambda b, c, lens: (0, 0)),
        ],
        out_specs=[
            pl.BlockSpec((1, 1, 1), lambda b, c, lens: (b, 0, 0),
                         memory_space=pltpu.SMEM),
            pl.BlockSpec((1, 1, 1), lambda b, c, lens: (b, 0, 0),
                         memory_space=pltpu.SMEM),
            pl.BlockSpec((1, 1, 1), lambda b, c, lens: (b, 0, 0),
                         memory_space=pltpu.SMEM),
        ],
        scratch_shapes=[
            pltpu.VMEM((K, NC), jnp.float32),
            pltpu.VMEM((1, H), jnp.float32),
        ],
    )
    act, lp, en = pl.pallas_call(
        _body,
        grid_spec=grid_spec,
        out_shape=[
            jax.ShapeDtypeStruct((B, 1, 1), jnp.int32),
            jax.ShapeDtypeStruct((B, 1, 1), jnp.float32),
            jax.ShapeDtypeStruct((B, 1, 1), jnp.float32),
        ],
    )(sentence_lengths, sequence_embedding, w1cat, b1r, w2c)
    return act[:, 0, 0], lp[:, 0, 0], en[:, 0, 0]


# speedup vs baseline: 1.2683x; 1.2673x over previous
import jax
import jax.numpy as jnp
from jax.experimental import pallas as pl
from jax.experimental.pallas import tpu as pltpu

_K = 1024


def _body(lens_ref, x_ref, w1_ref, b1_ref, w2_ref,
          act_ref, lp_ref, en_ref, sc_ref, carry_ref):
    b = pl.program_id(0)
    c = pl.program_id(1)
    nc = pl.num_programs(1)
    K, NC = sc_ref.shape
    Hh = carry_ref.shape[1]
    len_b = lens_ref[b]

    @pl.when(c * K <= len_b - 1)
    def _compute():
        x = x_ref[0].astype(jnp.bfloat16)
        r = jnp.dot(x, w1_ref[...], preferred_element_type=jnp.float32)
        p_part = r[:, :Hh]
        q_part = r[:, Hh:]
        p_shift = jnp.concatenate([carry_ref[...], p_part[:-1]], axis=0)
        h = jnp.maximum(p_shift + q_part + b1_ref[...], 0.0).astype(jnp.bfloat16)
        v = jnp.dot(h, w2_ref[...], preferred_element_type=jnp.float32)
        lane = jax.lax.broadcasted_iota(jnp.int32, (K, NC), 1)
        sc_ref[...] = jnp.where(lane == c, v, sc_ref[...])
        carry_ref[...] = p_part[-1:, :]

    @pl.when(c == nc - 1)
    def _finalize():
        s_all = sc_ref[...]
        g = (jax.lax.broadcasted_iota(jnp.int32, (K, NC), 0)
             + K * jax.lax.broadcasted_iota(jnp.int32, (K, NC), 1))
        valid = (g >= 1) & (g <= len_b - 1)
        s_m = jnp.where(valid, s_all, -jnp.inf)
        m = jnp.max(s_m)
        e = jnp.where(valid, jnp.exp(s_all - m), 0.0)
        l = jnp.sum(e)
        s_clean = jnp.where(valid, s_all, 0.0)
        t = jnp.sum(e * s_clean)
        cand = jnp.where(s_m == m, g, jnp.int32(2**30))
        gmin = jnp.min(cand)
        empty = len_b <= 1
        nan = jnp.float32(jnp.nan)
        logl = jnp.log(l)
        act_ref[0, 0, 0] = jnp.where(empty, 0, gmin - 1)
        lp_ref[0, 0, 0] = jnp.where(empty, nan, -logl)
        en_ref[0, 0, 0] = jnp.where(empty, nan, m + logl - t / l)


def kernel(sequence_embedding, sentence_lengths, W1, b1, W2, b2):
    B, S, D = sequence_embedding.shape
    H = W1.shape[0]
    K = _K
    NC = S // K
    w1cat = jnp.concatenate([W1[:, :D].T, W1[:, D:].T], axis=1).astype(jnp.bfloat16)
    b1r = b1.reshape(1, H)
    w2c = W2.reshape(H, 1).astype(jnp.bfloat16)

    grid_spec = pltpu.PrefetchScalarGridSpec(
        num_scalar_prefetch=1,
        grid=(B, NC),
        in_specs=[
            pl.BlockSpec(
                (1, K, D),
                lambda b, c, lens: (b, jnp.minimum(c, jnp.maximum(lens[b] - 1, 0) // K), 0)),
            pl.BlockSpec((D, 2 * H), lambda b, c, lens: (0, 0)),
            pl.BlockSpec((1, H), lambda b, c, lens: (0, 0)),
            pl.BlockSpec((H, 1), lambda b, c, lens: (0, 0)),
        ],
        out_specs=[
            pl.BlockSpec((1, 1, 1), lambda b, c, lens: (b, 0, 0),
                         memory_space=pltpu.SMEM),
            pl.BlockSpec((1, 1, 1), lambda b, c, lens: (b, 0, 0),
                         memory_space=pltpu.SMEM),
            pl.BlockSpec((1, 1, 1), lambda b, c, lens: (b, 0, 0),
                         memory_space=pltpu.SMEM),
        ],
        scratch_shapes=[
            pltpu.VMEM((K, NC), jnp.float32),
            pltpu.VMEM((1, H), jnp.float32),
        ],
    )
    act, lp, en = pl.pallas_call(
        _body,
        grid_spec=grid_spec,
        out_shape=[
            jax.ShapeDtypeStruct((B, 1, 1), jnp.int32),
            jax.ShapeDtypeStruct((B, 1, 1), jnp.float32),
            jax.ShapeDtypeStruct((B, 1, 1), jnp.float32),
        ],
    )(sentence_lengths, sequence_embedding, w1cat, b1r, w2c)
    return act[:, 0, 0], lp[:, 0, 0], en[:, 0, 0]
